# baseline (speedup 1.0000x reference)
import functools

import jax
import jax.numpy as jnp
from jax import lax
from jax.experimental import pallas as pl
from jax.experimental.pallas import tpu as pltpu
from jax.experimental.pallas import tpu_sc as plsc

N = 10000
NP = 10240
D = 128
E = 106666
NUM_RING = 3
CPT = 27
QT = CPT * 128
PER_CORE = 16 * QT
EP = 2 * PER_CORE
DUMP = N + 16
RPT = NP // 16

_MESH = plsc.VectorSubcoreMesh(core_axis_name="c", subcore_axis_name="s",
                               num_cores=2, num_subcores=16)


def _degree_body(src_hbm, dst_hbm, ones_hbm, z1_hbm,
                 out_o, out_i, acc_o, acc_i, ones_v, idx_s, idx_d, sem):
    c = lax.axis_index("c")
    s = lax.axis_index("s")
    row0 = s * RPT
    pltpu.sync_copy(ones_hbm, ones_v)
    pltpu.sync_copy(z1_hbm, acc_o.at[pl.ds(row0, RPT)])
    pltpu.sync_copy(z1_hbm, acc_i.at[pl.ds(row0, RPT)])
    for r in range(NUM_RING):
        pltpu.sync_copy(src_hbm.at[r, c, s], idx_s.at[r])
        pltpu.sync_copy(dst_hbm.at[r, c, s], idx_d.at[r])
    plsc.subcore_barrier()
    prev = None
    for r in range(NUM_RING):
        for k in range(CPT):
            cur = (pltpu.async_copy(ones_v, acc_o.at[idx_s.at[r, k]],
                                    sem, add=True),
                   pltpu.async_copy(ones_v, acc_i.at[idx_d.at[r, k]],
                                    sem, add=True))
            if prev is not None:
                prev[0].wait()
                prev[1].wait()
            prev = cur
    prev[0].wait()
    prev[1].wait()
    plsc.subcore_barrier()
    pltpu.sync_copy(acc_o.at[pl.ds(row0, RPT)], out_o.at[c, pl.ds(row0, RPT)])
    pltpu.sync_copy(acc_i.at[pl.ds(row0, RPT)], out_i.at[c, pl.ds(row0, RPT)])


_degree_kernel = pl.kernel(
    _degree_body,
    out_type=(
        jax.ShapeDtypeStruct((2, NP), jnp.float32),
        jax.ShapeDtypeStruct((2, NP), jnp.float32),
    ),
    mesh=_MESH,
    scratch_types=[
        pltpu.VMEM_SHARED((NP,), jnp.float32),
        pltpu.VMEM_SHARED((NP,), jnp.float32),
        pltpu.VMEM((128,), jnp.float32),
        pltpu.VMEM((NUM_RING, CPT, 128), jnp.int32),
        pltpu.VMEM((NUM_RING, CPT, 128), jnp.int32),
        pltpu.SemaphoreType.DMA,
    ],
)


def _agg_body(xn_hbm, src_hbm, dst_hbm, zrow_hbm,
              out, acc, idx_s, idx_d, rows, zbuf, sem0, sem1):
    c = lax.axis_index("c")
    s = lax.axis_index("s")
    row0 = s * RPT
    pltpu.sync_copy(zrow_hbm, zbuf)

    NBUF = 3
    LEAD = 2
    NCH = 2 * CPT

    def ring_body(r, carry):
        for zz in range(RPT // 64):
            pltpu.sync_copy(zbuf, acc.at[pl.ds(row0 + zz * 64, 64)])
        pltpu.sync_copy(src_hbm.at[r, c, s], idx_s)
        pltpu.sync_copy(dst_hbm.at[r, c, s], idx_d)
        plsc.subcore_barrier()

        gd = [None] * NCH
        sd = [None] * NCH
        for k in range(LEAD):
            gd[k] = pltpu.async_copy(xn_hbm.at[idx_d.at[k]],
                                     rows.at[k % NBUF], sem0)
        for k in range(NCH):
            p = k + LEAD
            if p < NCH:
                if p - NBUF >= 0:
                    sd[p - NBUF].wait()
                gd[p] = pltpu.async_copy(xn_hbm.at[idx_d.at[p]],
                                         rows.at[p % NBUF], sem0)
            gd[k].wait()
            sd[k] = pltpu.async_copy(rows.at[k % NBUF],
                                     acc.at[idx_s.at[k]], sem1, add=True)
        for k in range(NCH - NBUF, NCH):
            sd[k].wait()
        plsc.subcore_barrier()
        pltpu.sync_copy(acc.at[pl.ds(row0, RPT)],
                        out.at[r, c, pl.ds(row0, RPT)])
        return carry

    lax.fori_loop(0, NUM_RING, ring_body, 0)


_agg_kernel = pl.kernel(
    _agg_body,
    out_type=jax.ShapeDtypeStruct((NUM_RING, 2, NP, D), jnp.float32),
    mesh=_MESH,
    scratch_types=[
        pltpu.VMEM_SHARED((NP, D), jnp.float32),
        pltpu.VMEM((2 * CPT, 64), jnp.int32),
        pltpu.VMEM((2 * CPT, 64), jnp.int32),
        pltpu.VMEM((3, 64, D), jnp.float32),
        pltpu.VMEM((64, D), jnp.float32),
        pltpu.SemaphoreType.DMA,
        pltpu.SemaphoreType.DMA,
    ],
)


_BLK = 256


def _prep_body(x_ref, dego_ref, w_ref, xn_ref, zs_ref):
    deg = dego_ref[0, :] + dego_ref[1, :]
    onorm = lax.rsqrt(jnp.maximum(deg, 1.0))[:, None]
    x = x_ref[...]
    xn_ref[...] = x * onorm
    zs_ref[...] = lax.dot_general(
        x, w_ref[...], (((1,), (0,)), ((), ())),
        preferred_element_type=jnp.float32,
        precision=lax.Precision.HIGHEST)


def _prep_call(xp, dego, w_self):
    return pl.pallas_call(
        _prep_body,
        grid=(NP // _BLK,),
        in_specs=[
            pl.BlockSpec((_BLK, D), lambda i: (i, 0)),
            pl.BlockSpec((2, _BLK), lambda i: (0, i)),
            pl.BlockSpec((D, D), lambda i: (0, 0)),
        ],
        out_specs=[
            pl.BlockSpec((_BLK, D), lambda i: (i, 0)),
            pl.BlockSpec((_BLK, D), lambda i: (i, 0)),
        ],
        out_shape=[
            jax.ShapeDtypeStruct((NP, D), jnp.float32),
            jax.ShapeDtypeStruct((NP, D), jnp.float32),
        ],
    )(xp, dego, w_self)


def _mm(a, b):
    return lax.dot_general(a, b, (((1,), (0,)), ((), ())),
                           preferred_element_type=jnp.float32,
                           precision=lax.Precision.HIGHEST)


def _interaction(zh):
    gram = {}
    for i in range(4):
        for j in range(i, 4):
            gram[(i, j)] = jnp.sum(zh[i] * zh[j], axis=1, keepdims=True)
            gram[(j, i)] = gram[(i, j)]
    return gram


def _softmax_rows(scores):
    m = jnp.maximum(jnp.maximum(scores[0], scores[1]),
                    jnp.maximum(scores[2], scores[3]))
    es = [jnp.exp(sc - m) for sc in scores]
    tot = es[0] + es[1] + es[2] + es[3]
    inv = 1.0 / tot
    return [e * inv for e in es]


def _final_body(zs_ref, agg_ref, degi_ref, w0_ref, w1_ref, w2_ref,
                wc_ref, wd_ref, gw_ref, gb_ref, out_ref):
    degi = degi_ref[0, :] + degi_ref[1, :]
    inorm = lax.rsqrt(jnp.maximum(degi, 1.0))[:, None]
    z = [zs_ref[...] * inorm]
    for r, wr in enumerate((w0_ref, w1_ref, w2_ref)):
        a = agg_ref[r, 0] + agg_ref[r, 1]
        z.append(_mm(a, wr[...]) * inorm)

    wc = wc_ref[...]
    zc = [_mm(z[i], wc) for i in range(4)]
    gc = _interaction(zc)
    z_com = []
    for i in range(4):
        alpha = _softmax_rows([gc[(i, j)] for j in range(4)])
        acc = alpha[0] * zc[0]
        for j in range(1, 4):
            acc = acc + alpha[j] * zc[j]
        z_com.append(acc)

    wd = wd_ref[...]
    zd = [_mm(z[i], wd) for i in range(4)]
    gd = _interaction(zd)
    z_dis = []
    for i in range(4):
        alpha = _softmax_rows([gd[(i, i)] - gd[(i, j)] for j in range(4)])
        acc = alpha[0] * zd[0]
        for j in range(1, 4):
            acc = acc + alpha[j] * zd[j]
        z_dis.append(zd[i] - acc)

    gw = gw_ref[...]
    logit = gb_ref[0, 0]
    for i in range(4):
        logit = logit + jnp.sum(z_com[i] * gw[i * D:(i + 1) * D, 0][None, :],
                                axis=1, keepdims=True)
    for i in range(4):
        logit = logit + jnp.sum(
            z_dis[i] * gw[(4 + i) * D:(5 + i) * D, 0][None, :],
            axis=1, keepdims=True)
    beta = 1.0 / (1.0 + jnp.exp(-logit))
    for i in range(4):
        out_ref[:, i * D:(i + 1) * D] = (
            beta * z_com[i] + (1.0 - beta) * z_dis[i])


def _final_call(zs, agg, degi, w0, w1, w2, wc, wd, gw, gb):
    full = lambda *dims: pl.BlockSpec(dims, lambda i: tuple(0 for _ in dims))
    return pl.pallas_call(
        _final_body,
        grid=(NP // _BLK,),
        in_specs=[
            pl.BlockSpec((_BLK, D), lambda i: (i, 0)),
            pl.BlockSpec((NUM_RING, 2, _BLK, D), lambda i: (0, 0, i, 0)),
            pl.BlockSpec((2, _BLK), lambda i: (0, i)),
            full(D, D), full(D, D), full(D, D), full(D, D), full(D, D),
            full(8 * D, 1), full(1, 1),
        ],
        out_specs=pl.BlockSpec((_BLK, 4 * D), lambda i: (i, 0)),
        out_shape=jax.ShapeDtypeStruct((NP, 4 * D), jnp.float32),
    )(zs, agg, degi, w0, w1, w2, wc, wd, gw, gb)


def _pad_edges(ei):
    spread = DUMP + (jnp.arange(EP - E, dtype=jnp.int32) % (NP - DUMP))
    pad = jnp.broadcast_to(spread, (2, EP - E))
    e = jnp.concatenate([ei.astype(jnp.int32), pad], axis=1)
    src = e[0].reshape(2, 16, CPT, 128)
    dst = e[1].reshape(2, 16, CPT, 128)
    return src, dst


@jax.jit
def kernel(x, edge_index_r0, edge_index_r1, edge_index_r2,
           W_self, W_ring0, W_ring1, W_ring2, WC, WD, gate_W, gate_b):
    srcs, dsts = zip(*(_pad_edges(e)
                       for e in (edge_index_r0, edge_index_r1, edge_index_r2)))
    src_all = jnp.stack(srcs)
    dst_all = jnp.stack(dsts)

    ones1 = jnp.ones((128,), jnp.float32)
    z1 = jnp.zeros((RPT,), jnp.float32)
    zrow = jnp.zeros((64, D), jnp.float32)

    dego, degi = _degree_kernel(src_all, dst_all, ones1, z1)

    xp = jnp.zeros((NP, D), jnp.float32).at[:N].set(x)
    xn, zs = _prep_call(xp, dego, W_self)

    src64 = src_all.reshape(NUM_RING, 2, 16, 2 * CPT, 64)
    dst64 = dst_all.reshape(NUM_RING, 2, 16, 2 * CPT, 64)
    agg = _agg_kernel(xn, src64, dst64, zrow)

    out = _final_call(zs, agg, degi, W_ring0, W_ring1, W_ring2,
                      WC, WD, gate_W, gate_b.reshape(1, 1))
    return out[:N]


# speedup vs baseline: 1.0057x; 1.0057x over previous
import jax
import jax.numpy as jnp
from jax import lax
from jax.experimental import pallas as pl
from jax.experimental.pallas import tpu as pltpu
from jax.experimental.pallas import tpu_sc as plsc

N = 10000
NP = 10240
D = 128
E = 106666
NUM_RING = 3
CPT = 27
QT = CPT * 128
PER_CORE = 16 * QT
EP = 2 * PER_CORE
DUMP = N + 16
RPT = NP // 16

_MESH = plsc.VectorSubcoreMesh(core_axis_name="c", subcore_axis_name="s",
                               num_cores=2, num_subcores=16)


def _degree_body(src_hbm, dst_hbm, ones_hbm, z1_hbm,
                 out_o, out_i, acc_o, acc_i, ones_v, idx_s, idx_d, sem):
    c = lax.axis_index("c")
    s = lax.axis_index("s")
    row0 = s * RPT
    pltpu.sync_copy(ones_hbm, ones_v)
    pltpu.sync_copy(z1_hbm, acc_o.at[pl.ds(row0, RPT)])
    pltpu.sync_copy(z1_hbm, acc_i.at[pl.ds(row0, RPT)])
    for r in range(NUM_RING):
        pltpu.sync_copy(src_hbm.at[r, c, s], idx_s.at[r])
        pltpu.sync_copy(dst_hbm.at[r, c, s], idx_d.at[r])
    plsc.subcore_barrier()
    prev = None
    for r in range(NUM_RING):
        for k in range(CPT):
            cur = (pltpu.async_copy(ones_v, acc_o.at[idx_s.at[r, k]],
                                    sem, add=True),
                   pltpu.async_copy(ones_v, acc_i.at[idx_d.at[r, k]],
                                    sem, add=True))
            if prev is not None:
                prev[0].wait()
                prev[1].wait()
            prev = cur
    prev[0].wait()
    prev[1].wait()
    plsc.subcore_barrier()
    pltpu.sync_copy(acc_o.at[pl.ds(row0, RPT)], out_o.at[c, pl.ds(row0, RPT)])
    pltpu.sync_copy(acc_i.at[pl.ds(row0, RPT)], out_i.at[c, pl.ds(row0, RPT)])


_degree_kernel = pl.kernel(
    _degree_body,
    out_type=(
        jax.ShapeDtypeStruct((2, NP), jnp.float32),
        jax.ShapeDtypeStruct((2, NP), jnp.float32),
    ),
    mesh=_MESH,
    scratch_types=[
        pltpu.VMEM_SHARED((NP,), jnp.float32),
        pltpu.VMEM_SHARED((NP,), jnp.float32),
        pltpu.VMEM((128,), jnp.float32),
        pltpu.VMEM((NUM_RING, CPT, 128), jnp.int32),
        pltpu.VMEM((NUM_RING, CPT, 128), jnp.int32),
        pltpu.SemaphoreType.DMA,
    ],
)


def _agg_body(xn_hbm, src_hbm, dst_hbm, zrow_hbm,
              out, acc, idx_s, idx_d, rows, zbuf, sem0, sem1):
    c = lax.axis_index("c")
    s = lax.axis_index("s")
    row0 = s * RPT
    pltpu.sync_copy(zrow_hbm, zbuf)

    NBUF = 2
    LEAD = 1
    NCH = CPT

    def ring_body(r, carry):
        for zz in range(RPT // 64):
            pltpu.sync_copy(zbuf, acc.at[pl.ds(row0 + zz * 64, 64)])
        pltpu.sync_copy(src_hbm.at[r, c, s], idx_s)
        pltpu.sync_copy(dst_hbm.at[r, c, s], idx_d)
        plsc.subcore_barrier()

        gd = [None] * NCH
        sd = [None] * NCH
        for k in range(LEAD):
            gd[k] = pltpu.async_copy(xn_hbm.at[idx_d.at[k]],
                                     rows.at[k % NBUF], sem0)
        for k in range(NCH):
            p = k + LEAD
            if p < NCH:
                if p - NBUF >= 0:
                    sd[p - NBUF].wait()
                gd[p] = pltpu.async_copy(xn_hbm.at[idx_d.at[p]],
                                         rows.at[p % NBUF], sem0)
            gd[k].wait()
            sd[k] = pltpu.async_copy(rows.at[k % NBUF],
                                     acc.at[idx_s.at[k]], sem1, add=True)
        for k in range(NCH - NBUF, NCH):
            sd[k].wait()
        plsc.subcore_barrier()
        pltpu.sync_copy(acc.at[pl.ds(row0, RPT)],
                        out.at[r, c, pl.ds(row0, RPT)])
        return carry

    lax.fori_loop(0, NUM_RING, ring_body, 0)


_agg_kernel = pl.kernel(
    _agg_body,
    out_type=jax.ShapeDtypeStruct((NUM_RING, 2, NP, D), jnp.float32),
    mesh=_MESH,
    scratch_types=[
        pltpu.VMEM_SHARED((NP, D), jnp.float32),
        pltpu.VMEM((CPT, 128), jnp.int32),
        pltpu.VMEM((CPT, 128), jnp.int32),
        pltpu.VMEM((2, 128, D), jnp.float32),
        pltpu.VMEM((64, D), jnp.float32),
        pltpu.SemaphoreType.DMA,
        pltpu.SemaphoreType.DMA,
    ],
)


_BLK = 256


def _prep_body(x_ref, dego_ref, w_ref, xn_ref, zs_ref):
    deg = dego_ref[0, :] + dego_ref[1, :]
    onorm = lax.rsqrt(jnp.maximum(deg, 1.0))[:, None]
    x = x_ref[...]
    xn_ref[...] = x * onorm
    zs_ref[...] = lax.dot_general(
        x, w_ref[...], (((1,), (0,)), ((), ())),
        preferred_element_type=jnp.float32,
        precision=lax.Precision.HIGHEST)


def _prep_call(xp, dego, w_self):
    return pl.pallas_call(
        _prep_body,
        grid=(NP // _BLK,),
        in_specs=[
            pl.BlockSpec((_BLK, D), lambda i: (i, 0)),
            pl.BlockSpec((2, _BLK), lambda i: (0, i)),
            pl.BlockSpec((D, D), lambda i: (0, 0)),
        ],
        out_specs=[
            pl.BlockSpec((_BLK, D), lambda i: (i, 0)),
            pl.BlockSpec((_BLK, D), lambda i: (i, 0)),
        ],
        out_shape=[
            jax.ShapeDtypeStruct((NP, D), jnp.float32),
            jax.ShapeDtypeStruct((NP, D), jnp.float32),
        ],
    )(xp, dego, w_self)


def _mm(a, b):
    return lax.dot_general(a, b, (((1,), (0,)), ((), ())),
                           preferred_element_type=jnp.float32,
                           precision=lax.Precision.HIGHEST)


def _interaction(zh):
    gram = {}
    for i in range(4):
        for j in range(i, 4):
            gram[(i, j)] = jnp.sum(zh[i] * zh[j], axis=1, keepdims=True)
            gram[(j, i)] = gram[(i, j)]
    return gram


def _softmax_rows(scores):
    m = jnp.maximum(jnp.maximum(scores[0], scores[1]),
                    jnp.maximum(scores[2], scores[3]))
    es = [jnp.exp(sc - m) for sc in scores]
    tot = es[0] + es[1] + es[2] + es[3]
    inv = 1.0 / tot
    return [e * inv for e in es]


def _final_body(zs_ref, agg_ref, degi_ref, w0_ref, w1_ref, w2_ref,
                wc_ref, wd_ref, gw_ref, gb_ref, out_ref):
    degi = degi_ref[0, :] + degi_ref[1, :]
    inorm = lax.rsqrt(jnp.maximum(degi, 1.0))[:, None]
    z = [zs_ref[...] * inorm]
    for r, wr in enumerate((w0_ref, w1_ref, w2_ref)):
        a = agg_ref[r, 0] + agg_ref[r, 1]
        z.append(_mm(a, wr[...]) * inorm)

    wc = wc_ref[...]
    zc = [_mm(z[i], wc) for i in range(4)]
    gc = _interaction(zc)
    z_com = []
    for i in range(4):
        alpha = _softmax_rows([gc[(i, j)] for j in range(4)])
        acc = alpha[0] * zc[0]
        for j in range(1, 4):
            acc = acc + alpha[j] * zc[j]
        z_com.append(acc)

    wd = wd_ref[...]
    zd = [_mm(z[i], wd) for i in range(4)]
    gd = _interaction(zd)
    z_dis = []
    for i in range(4):
        alpha = _softmax_rows([gd[(i, i)] - gd[(i, j)] for j in range(4)])
        acc = alpha[0] * zd[0]
        for j in range(1, 4):
            acc = acc + alpha[j] * zd[j]
        z_dis.append(zd[i] - acc)

    gw = gw_ref[...]
    logit = gb_ref[0, 0]
    for i in range(4):
        logit = logit + jnp.sum(z_com[i] * gw[i * D:(i + 1) * D, 0][None, :],
                                axis=1, keepdims=True)
    for i in range(4):
        logit = logit + jnp.sum(
            z_dis[i] * gw[(4 + i) * D:(5 + i) * D, 0][None, :],
            axis=1, keepdims=True)
    beta = 1.0 / (1.0 + jnp.exp(-logit))
    for i in range(4):
        out_ref[:, i * D:(i + 1) * D] = (
            beta * z_com[i] + (1.0 - beta) * z_dis[i])


def _final_call(zs, agg, degi, w0, w1, w2, wc, wd, gw, gb):
    full = lambda *dims: pl.BlockSpec(dims, lambda i: tuple(0 for _ in dims))
    return pl.pallas_call(
        _final_body,
        grid=(NP // _BLK,),
        in_specs=[
            pl.BlockSpec((_BLK, D), lambda i: (i, 0)),
            pl.BlockSpec((NUM_RING, 2, _BLK, D), lambda i: (0, 0, i, 0)),
            pl.BlockSpec((2, _BLK), lambda i: (0, i)),
            full(D, D), full(D, D), full(D, D), full(D, D), full(D, D),
            full(8 * D, 1), full(1, 1),
        ],
        out_specs=pl.BlockSpec((_BLK, 4 * D), lambda i: (i, 0)),
        out_shape=jax.ShapeDtypeStruct((NP, 4 * D), jnp.float32),
    )(zs, agg, degi, w0, w1, w2, wc, wd, gw, gb)


def _pad_edges(ei):
    spread = DUMP + (jnp.arange(EP - E, dtype=jnp.int32) % (NP - DUMP))
    pad = jnp.broadcast_to(spread, (2, EP - E))
    e = jnp.concatenate([ei.astype(jnp.int32), pad], axis=1)
    src = e[0].reshape(2, 16, CPT, 128)
    dst = e[1].reshape(2, 16, CPT, 128)
    return src, dst


@jax.jit
def kernel(x, edge_index_r0, edge_index_r1, edge_index_r2,
           W_self, W_ring0, W_ring1, W_ring2, WC, WD, gate_W, gate_b):
    srcs, dsts = zip(*(_pad_edges(e)
                       for e in (edge_index_r0, edge_index_r1, edge_index_r2)))
    src_all = jnp.stack(srcs)
    dst_all = jnp.stack(dsts)

    ones1 = jnp.ones((128,), jnp.float32)
    z1 = jnp.zeros((RPT,), jnp.float32)
    zrow = jnp.zeros((64, D), jnp.float32)

    dego, degi = _degree_kernel(src_all, dst_all, ones1, z1)

    xp = jnp.zeros((NP, D), jnp.float32).at[:N].set(x)
    xn, zs = _prep_call(xp, dego, W_self)

    agg = _agg_kernel(xn, src_all, dst_all, zrow)

    out = _final_call(zs, agg, degi, W_ring0, W_ring1, W_ring2,
                      WC, WD, gate_W, gate_b.reshape(1, 1))
    return out[:N]
